# baseline (speedup 1.0000x reference)
import jax
import jax.numpy as jnp
from jax import lax
from jax.experimental import pallas as pl
from jax.experimental.pallas import tpu as pltpu
from jax.experimental.pallas import tpu_sc as plsc

N = 10000
D = 128
E = 320000
NC = 2
NS = 16
TILES = NC * NS
CHUNK = 128
GROUP = 8
E_PAD = 327680
NCHUNK1 = E_PAD // NS // CHUNK
NGROUP1 = NCHUNK1 // GROUP
NCHUNK2 = E_PAD // TILES // CHUNK
NGROUP2 = NCHUNK2 // GROUP
NPAD = 10240
RZ = NPAD // NS


def _gs_group(xr, acc, srcv, dstv, rows_a, rows_b, sem_a, sem_b):
    bufs = [(rows_a, sem_a), (rows_b, sem_b)]

    def cp(j):
        buf, sem = bufs[j % 2]
        return pltpu.make_async_copy(xr.at[srcv.at[j]], buf, sem)

    cp(0).start()
    for j in range(GROUP):
        cp(j).wait()
        if j + 1 < GROUP:
            cp(j + 1).start()
        pltpu.sync_copy(bufs[j % 2][0], acc.at[dstv.at[j]], add=True)


def _agg1_body(xr, srcr, dstr, zr, onesr,
               out_s, out_c,
               acc, srcv, dstv, rows_a, rows_b, sem_a, sem_b):
    c = lax.axis_index("c")
    s = lax.axis_index("s")
    pltpu.sync_copy(zr, acc.at[pl.ds(s * RZ, RZ)])
    plsc.subcore_barrier()

    @pl.when(c == 0)
    def _sums():
        def group(g, carry):
            pltpu.sync_copy(srcr.at[s, pl.ds(g * GROUP, GROUP)], srcv)
            pltpu.sync_copy(dstr.at[s, pl.ds(g * GROUP, GROUP)], dstv)
            _gs_group(xr, acc, srcv, dstv, rows_a, rows_b, sem_a, sem_b)
            return carry

        lax.fori_loop(0, NGROUP1, group, 0)

    @pl.when(c == 1)
    def _counts():
        pltpu.sync_copy(onesr, rows_a)

        def group(g, carry):
            pltpu.sync_copy(dstr.at[s, pl.ds(g * GROUP, GROUP)], dstv)

            def chunk(j, carry2):
                pltpu.sync_copy(rows_a, acc.at[dstv.at[j]], add=True)
                return carry2

            lax.fori_loop(0, GROUP, chunk, 0)
            return carry

        lax.fori_loop(0, NGROUP1, group, 0)

    plsc.subcore_barrier()

    @pl.when(c == 0)
    def _wr_s():
        pltpu.sync_copy(acc.at[pl.ds(s * RZ, RZ)], out_s.at[pl.ds(s * RZ, RZ)])

    @pl.when(c == 1)
    def _wr_c():
        pltpu.sync_copy(acc.at[pl.ds(s * RZ, RZ)], out_c.at[pl.ds(s * RZ, RZ)])


def _agg2_body(xr, srcr, dstr, zr,
               out_s,
               acc, srcv, dstv, rows_a, rows_b, sem_a, sem_b):
    c = lax.axis_index("c")
    s = lax.axis_index("s")
    wid = s * NC + c
    pltpu.sync_copy(zr, acc.at[pl.ds(s * RZ, RZ)])
    plsc.subcore_barrier()

    def group(g, carry):
        pltpu.sync_copy(srcr.at[wid, pl.ds(g * GROUP, GROUP)], srcv)
        pltpu.sync_copy(dstr.at[wid, pl.ds(g * GROUP, GROUP)], dstv)
        _gs_group(xr, acc, srcv, dstv, rows_a, rows_b, sem_a, sem_b)
        return carry

    lax.fori_loop(0, NGROUP2, group, 0)
    plsc.subcore_barrier()
    pltpu.sync_copy(acc.at[pl.ds(s * RZ, RZ)], out_s.at[c, pl.ds(s * RZ, RZ)])


import functools


@functools.lru_cache(maxsize=None)
def _make_aggs():
    mesh = plsc.VectorSubcoreMesh(core_axis_name="c", subcore_axis_name="s",
                                  num_cores=NC, num_subcores=NS)
    agg1 = pl.kernel(
        _agg1_body,
        out_type=[jax.ShapeDtypeStruct((NPAD, D), jnp.float32),
                  jax.ShapeDtypeStruct((NPAD, D), jnp.float32)],
        mesh=mesh,
        scratch_types=[pltpu.VMEM_SHARED((NPAD, D), jnp.float32),
                       pltpu.VMEM((GROUP, CHUNK), jnp.int32),
                       pltpu.VMEM((GROUP, CHUNK), jnp.int32),
                       pltpu.VMEM((CHUNK, D), jnp.float32),
                       pltpu.VMEM((CHUNK, D), jnp.float32),
                       pltpu.SemaphoreType.DMA,
                       pltpu.SemaphoreType.DMA])
    agg2 = pl.kernel(
        _agg2_body,
        out_type=[jax.ShapeDtypeStruct((NC, NPAD, D), jnp.float32)],
        mesh=mesh,
        scratch_types=[pltpu.VMEM_SHARED((NPAD, D), jnp.float32),
                       pltpu.VMEM((GROUP, CHUNK), jnp.int32),
                       pltpu.VMEM((GROUP, CHUNK), jnp.int32),
                       pltpu.VMEM((CHUNK, D), jnp.float32),
                       pltpu.VMEM((CHUNK, D), jnp.float32),
                       pltpu.SemaphoreType.DMA,
                       pltpu.SemaphoreType.DMA])
    return agg1, agg2


_BLK = 1000
_GRID = N // _BLK


def _stage1_body(s_ref, c_ref, x_ref, wl_ref, b_ref, wr_ref, o_ref):
    cnt = c_ref[:, 0:1]
    mean = s_ref[...] / jnp.maximum(cnt, 1.0)
    h = (jnp.dot(mean, wl_ref[...], preferred_element_type=jnp.float32)
         + b_ref[...]
         + jnp.dot(x_ref[...], wr_ref[...], preferred_element_type=jnp.float32))
    o_ref[...] = jnp.maximum(h, 0.0)


def _stage2_body(p_ref, c_ref, h_ref, wl2_ref, b2_ref, wr2_ref,
                 wl3_ref, b3_ref, wr3_ref, eps_ref, o_ref):
    ssum = p_ref[0] + p_ref[1]
    cnt = c_ref[:, 0:1]
    mean = ssum / jnp.maximum(cnt, 1.0)
    h = h_ref[...]
    mu = (jnp.dot(mean, wl2_ref[...], preferred_element_type=jnp.float32)
          + b2_ref[...]
          + jnp.dot(h, wr2_ref[...], preferred_element_type=jnp.float32))
    ls = (jnp.dot(mean, wl3_ref[...], preferred_element_type=jnp.float32)
          + b3_ref[...]
          + jnp.dot(h, wr3_ref[...], preferred_element_type=jnp.float32))
    ls = jnp.minimum(ls, 10.0)
    o_ref[...] = mu + eps_ref[...] * jnp.exp(ls)


def _row_spec():
    return pl.BlockSpec((_BLK, D), lambda i: (i, 0))


def _full_spec(shape):
    return pl.BlockSpec(shape, lambda i: tuple(0 for _ in shape))


_stage1 = pl.pallas_call(
    _stage1_body,
    grid=(_GRID,),
    in_specs=[_row_spec(), _row_spec(),
              _row_spec(), _full_spec((D, D)), _full_spec((1, D)),
              _full_spec((D, D))],
    out_specs=_row_spec(),
    out_shape=jax.ShapeDtypeStruct((N, D), jnp.float32),
)

_stage2 = pl.pallas_call(
    _stage2_body,
    grid=(_GRID,),
    in_specs=[pl.BlockSpec((NC, _BLK, D), lambda i: (0, i, 0)),
              _row_spec(),
              _row_spec(), _full_spec((D, D)), _full_spec((1, D)),
              _full_spec((D, D)), _full_spec((D, D)), _full_spec((1, D)),
              _full_spec((D, D)), _row_spec()],
    out_specs=_row_spec(),
    out_shape=jax.ShapeDtypeStruct((N, D), jnp.float32),
)


@jax.jit
def kernel(x, edge_index, edge_weight, Wl1, bl1, Wr1, Wl2, bl2, Wr2, Wl3, bl3, Wr3):
    src = edge_index[0]
    dst = edge_index[1]
    pad = E_PAD - E
    src_flat = jnp.concatenate([src, jnp.zeros((pad,), jnp.int32)])
    dst_flat = jnp.concatenate([dst, jnp.full((pad,), N, jnp.int32)])
    src1 = src_flat.reshape(NS, NCHUNK1, CHUNK)
    dst1 = dst_flat.reshape(NS, NCHUNK1, CHUNK)
    src2 = src_flat.reshape(TILES, NCHUNK2, CHUNK)
    dst2 = dst_flat.reshape(TILES, NCHUNK2, CHUNK)
    zr = jnp.zeros((RZ, D), jnp.float32)
    onesr = jnp.ones((CHUNK, D), jnp.float32)

    agg1, agg2 = _make_aggs()
    sums1, cnts = agg1(x, src1, dst1, zr, onesr)
    h = _stage1(sums1, cnts, x, Wl1.T, bl1.reshape(1, D), Wr1.T)
    (sums2,) = agg2(h, src2, dst2, zr)
    eps = jax.random.normal(jax.random.key(42), (N, D), dtype=jnp.float32)
    z = _stage2(sums2, cnts, h, Wl2.T, bl2.reshape(1, D), Wr2.T,
                Wl3.T, bl3.reshape(1, D), Wr3.T, eps)
    return z


# speedup vs baseline: 1.0136x; 1.0136x over previous
import jax
import jax.numpy as jnp
from jax import lax
from jax.experimental import pallas as pl
from jax.experimental.pallas import tpu as pltpu
from jax.experimental.pallas import tpu_sc as plsc

N = 10000
D = 128
E = 320000
NC = 2
NS = 16
TILES = NC * NS
CHUNK = 128
GROUP = 8
E_PAD = 327680
NCHUNK = E_PAD // TILES // CHUNK
NGROUP = NCHUNK // GROUP
NPAD = 10240
RZ = NPAD // NS


def _gs_group(xr, acc, srcv, dstv, rows_a, rows_b, sem_a, sem_b):
    bufs = [(rows_a, sem_a), (rows_b, sem_b)]

    def cp(j):
        buf, sem = bufs[j % 2]
        return pltpu.make_async_copy(xr.at[srcv.at[j]], buf, sem)

    cp(0).start()
    for j in range(GROUP):
        cp(j).wait()
        if j + 1 < GROUP:
            cp(j + 1).start()
        pltpu.sync_copy(bufs[j % 2][0], acc.at[dstv.at[j]], add=True)


def _sum_phase(xr, srcr, dstr, acc, wid, srcv, dstv, rows_a, rows_b,
               sem_a, sem_b):
    def group(g, carry):
        pltpu.sync_copy(srcr.at[wid, pl.ds(g * GROUP, GROUP)], srcv)
        pltpu.sync_copy(dstr.at[wid, pl.ds(g * GROUP, GROUP)], dstv)
        _gs_group(xr, acc, srcv, dstv, rows_a, rows_b, sem_a, sem_b)
        return carry

    lax.fori_loop(0, NGROUP, group, 0)


def _agg1_body(xr, srcr, dstr, zr, onesr,
               out_s, out_c,
               acc, srcv, dstv, rows_a, rows_b, sem_a, sem_b):
    c = lax.axis_index("c")
    s = lax.axis_index("s")
    wid = s * NC + c

    pltpu.sync_copy(zr, acc.at[pl.ds(s * RZ, RZ)])
    plsc.subcore_barrier()
    _sum_phase(xr, srcr, dstr, acc, wid, srcv, dstv, rows_a, rows_b,
               sem_a, sem_b)
    plsc.subcore_barrier()
    pltpu.sync_copy(acc.at[pl.ds(s * RZ, RZ)], out_s.at[c, pl.ds(s * RZ, RZ)])

    pltpu.sync_copy(zr, acc.at[pl.ds(s * RZ, RZ)])
    plsc.subcore_barrier()
    pltpu.sync_copy(onesr, rows_a)

    def group(g, carry):
        pltpu.sync_copy(dstr.at[wid, pl.ds(g * GROUP, GROUP)], dstv)

        def chunk(j, carry2):
            pltpu.sync_copy(rows_a, acc.at[dstv.at[j]], add=True)
            return carry2

        lax.fori_loop(0, GROUP, chunk, 0)
        return carry

    lax.fori_loop(0, NGROUP, group, 0)
    plsc.subcore_barrier()
    pltpu.sync_copy(acc.at[pl.ds(s * RZ, RZ)], out_c.at[c, pl.ds(s * RZ, RZ)])


def _agg2_body(xr, srcr, dstr, zr,
               out_s,
               acc, srcv, dstv, rows_a, rows_b, sem_a, sem_b):
    c = lax.axis_index("c")
    s = lax.axis_index("s")
    wid = s * NC + c
    pltpu.sync_copy(zr, acc.at[pl.ds(s * RZ, RZ)])
    plsc.subcore_barrier()
    _sum_phase(xr, srcr, dstr, acc, wid, srcv, dstv, rows_a, rows_b,
               sem_a, sem_b)
    plsc.subcore_barrier()
    pltpu.sync_copy(acc.at[pl.ds(s * RZ, RZ)], out_s.at[c, pl.ds(s * RZ, RZ)])


import functools


@functools.lru_cache(maxsize=None)
def _make_aggs():
    mesh = plsc.VectorSubcoreMesh(core_axis_name="c", subcore_axis_name="s",
                                  num_cores=NC, num_subcores=NS)
    scratch = [pltpu.VMEM_SHARED((NPAD, D), jnp.float32),
               pltpu.VMEM((GROUP, CHUNK), jnp.int32),
               pltpu.VMEM((GROUP, CHUNK), jnp.int32),
               pltpu.VMEM((CHUNK, D), jnp.float32),
               pltpu.VMEM((CHUNK, D), jnp.float32),
               pltpu.SemaphoreType.DMA,
               pltpu.SemaphoreType.DMA]
    agg1 = pl.kernel(
        _agg1_body,
        out_type=[jax.ShapeDtypeStruct((NC, NPAD, D), jnp.float32),
                  jax.ShapeDtypeStruct((NC, NPAD, D), jnp.float32)],
        mesh=mesh,
        scratch_types=list(scratch))
    agg2 = pl.kernel(
        _agg2_body,
        out_type=[jax.ShapeDtypeStruct((NC, NPAD, D), jnp.float32)],
        mesh=mesh,
        scratch_types=list(scratch))
    return agg1, agg2


_BLK = 1000
_GRID = N // _BLK


def _stage1_body(p_ref, c_ref, x_ref, wl_ref, b_ref, wr_ref, o_ref):
    ssum = p_ref[0] + p_ref[1]
    cnt = (c_ref[0] + c_ref[1])[:, 0:1]
    mean = ssum / jnp.maximum(cnt, 1.0)
    h = (jnp.dot(mean, wl_ref[...], preferred_element_type=jnp.float32)
         + b_ref[...]
         + jnp.dot(x_ref[...], wr_ref[...], preferred_element_type=jnp.float32))
    o_ref[...] = jnp.maximum(h, 0.0)


def _stage2_body(p_ref, c_ref, h_ref, wl2_ref, b2_ref, wr2_ref,
                 wl3_ref, b3_ref, wr3_ref, eps_ref, o_ref):
    ssum = p_ref[0] + p_ref[1]
    cnt = (c_ref[0] + c_ref[1])[:, 0:1]
    mean = ssum / jnp.maximum(cnt, 1.0)
    h = h_ref[...]
    mu = (jnp.dot(mean, wl2_ref[...], preferred_element_type=jnp.float32)
          + b2_ref[...]
          + jnp.dot(h, wr2_ref[...], preferred_element_type=jnp.float32))
    ls = (jnp.dot(mean, wl3_ref[...], preferred_element_type=jnp.float32)
          + b3_ref[...]
          + jnp.dot(h, wr3_ref[...], preferred_element_type=jnp.float32))
    ls = jnp.minimum(ls, 10.0)
    o_ref[...] = mu + eps_ref[...] * jnp.exp(ls)


def _row_spec():
    return pl.BlockSpec((_BLK, D), lambda i: (i, 0))


def _part_spec():
    return pl.BlockSpec((NC, _BLK, D), lambda i: (0, i, 0))


def _full_spec(shape):
    return pl.BlockSpec(shape, lambda i: tuple(0 for _ in shape))


_stage1 = pl.pallas_call(
    _stage1_body,
    grid=(_GRID,),
    in_specs=[_part_spec(), _part_spec(),
              _row_spec(), _full_spec((D, D)), _full_spec((1, D)),
              _full_spec((D, D))],
    out_specs=_row_spec(),
    out_shape=jax.ShapeDtypeStruct((N, D), jnp.float32),
)

_stage2 = pl.pallas_call(
    _stage2_body,
    grid=(_GRID,),
    in_specs=[_part_spec(), _part_spec(),
              _row_spec(), _full_spec((D, D)), _full_spec((1, D)),
              _full_spec((D, D)), _full_spec((D, D)), _full_spec((1, D)),
              _full_spec((D, D)), _row_spec()],
    out_specs=_row_spec(),
    out_shape=jax.ShapeDtypeStruct((N, D), jnp.float32),
)


@jax.jit
def kernel(x, edge_index, edge_weight, Wl1, bl1, Wr1, Wl2, bl2, Wr2, Wl3, bl3, Wr3):
    src = edge_index[0]
    dst = edge_index[1]
    pad = E_PAD - E
    src_flat = jnp.concatenate([src, jnp.zeros((pad,), jnp.int32)])
    dst_flat = jnp.concatenate([dst, jnp.full((pad,), N, jnp.int32)])
    src2 = src_flat.reshape(TILES, NCHUNK, CHUNK)
    dst2 = dst_flat.reshape(TILES, NCHUNK, CHUNK)
    zr = jnp.zeros((RZ, D), jnp.float32)
    onesr = jnp.ones((CHUNK, D), jnp.float32)

    agg1, agg2 = _make_aggs()
    sums1, cnts = agg1(x, src2, dst2, zr, onesr)
    h = _stage1(sums1, cnts, x, Wl1.T, bl1.reshape(1, D), Wr1.T)
    (sums2,) = agg2(h, src2, dst2, zr)
    eps = jax.random.normal(jax.random.key(42), (N, D), dtype=jnp.float32)
    z = _stage2(sums2, cnts, h, Wl2.T, bl2.reshape(1, D), Wr2.T,
                Wl3.T, bl3.reshape(1, D), Wr3.T, eps)
    return z
